# speedup vs baseline: 1.1337x; 1.1337x over previous
import functools

import jax
import jax.numpy as jnp
import numpy as np
from jax import lax
from jax.experimental import pallas as pl
from jax.experimental.pallas import tpu as pltpu

N_NODES = 10000
D = 256
DH = 128


_ROWS_B = 1000


def _proj_body(x_ref, w_ref, att_ref, h_ref, apair_ref, amax_ref):
    half = pl.program_id(1)
    hb = jnp.dot(x_ref[...], w_ref[...], preferred_element_type=jnp.float32)
    h_ref[0] = hb
    ap = jnp.dot(hb, att_ref[...], preferred_element_type=jnp.float32)

    @pl.when(half == 0)
    def _():
        apair_ref[...] = ap

    @pl.when(half == 1)
    def _():
        acc = apair_ref[...] + ap
        apair_ref[...] = acc
        amax_ref[0, 0, :] = jnp.max(acc, axis=0)


def _projection(features, W, att2):
    n = features.shape[0]
    nb = n // _ROWS_B
    return pl.pallas_call(
        _proj_body,
        grid=(nb, 2),
        in_specs=[
            pl.BlockSpec((_ROWS_B, D), lambda i, h: (i, 0)),
            pl.BlockSpec((D, DH), lambda i, h: (0, h)),
            pl.BlockSpec((DH, 2), lambda i, h: (h, 0)),
        ],
        out_specs=[
            pl.BlockSpec((1, _ROWS_B, DH), lambda i, h: (h, i, 0)),
            pl.BlockSpec((_ROWS_B, 2), lambda i, h: (i, 0)),
            pl.BlockSpec((1, 1, 2), lambda i, h: (i, 0, 0)),
        ],
        out_shape=[
            jax.ShapeDtypeStruct((2, n, DH), jnp.float32),
            jax.ShapeDtypeStruct((n, 2), jnp.float32),
            jax.ShapeDtypeStruct((nb, 1, 2), jnp.float32),
        ],
    )(features, W, att2)


def kernel(features, edge_index, pos_edge_index, neg_edge_index,
           neg_sample_size, W, att_src, att_dst, bias):
    n = features.shape[0]
    att2 = jnp.stack([att_src, att_dst], axis=1)
    h_pair, apair, amax = _projection(features, W, att2)

    mx = jnp.max(amax, axis=(0, 1))
    msum = mx[0] + mx[1]
    M = jnp.where(msum > 0, msum, 0.2 * msum)

    h = jnp.concatenate([h_pair[0], h_pair[1]], axis=1)
    a_src = apair[:, 0]
    a_dst = apair[:, 1]

    src = edge_index[0]
    dst = edge_index[1]
    e = a_src[src] + a_dst[dst]
    e = jnp.where(e > 0, e, 0.2 * e)
    ex = jnp.exp(e - M)
    denom = jax.ops.segment_sum(ex, dst, num_segments=n)
    alpha = ex / (denom[dst] + 1e-16)
    msgs = alpha[:, None] * h[src]
    out = jax.ops.segment_sum(msgs, dst, num_segments=n) + bias
    emb = jnp.where(out > 0, out, jnp.exp(jnp.minimum(out, 0.0)) - 1.0)

    def score(eidx):
        return jnp.sum(emb[eidx[0]] * emb[eidx[1]], axis=1)

    pos_score = score(pos_edge_index)
    neg_score = score(neg_edge_index)
    pos_l = jax.nn.log_sigmoid(pos_score)
    neg_l = jax.nn.log_sigmoid(-neg_score)
    loss = -(jnp.sum(pos_l) + jnp.sum(neg_l)) / pos_score.shape[0]
    return emb, loss


# speedup vs baseline: 1.6230x; 1.4316x over previous
import functools

import jax
import jax.numpy as jnp
import numpy as np
from jax import lax
from jax.experimental import pallas as pl
from jax.experimental.pallas import tpu as pltpu
from jax.experimental.pallas import tpu_sc as plsc

N_NODES = 10000
D = 256
DH = 128

NC = 2
NS = 16
NW = NC * NS

NP = 10240
E_PAD = 161792
EW_A = E_PAD // NW


_ROWS_B = 1000


def _proj_body(x_ref, w_ref, att_ref, h_ref, apair_ref, amax_ref):
    half = pl.program_id(1)
    hb = jnp.dot(x_ref[...], w_ref[...], preferred_element_type=jnp.float32)
    h_ref[0] = hb
    ap = jnp.dot(hb, att_ref[...], preferred_element_type=jnp.float32)

    @pl.when(half == 0)
    def _():
        apair_ref[...] = ap

    @pl.when(half == 1)
    def _():
        acc = apair_ref[...] + ap
        apair_ref[...] = acc
        amax_ref[0, 0, :] = jnp.max(acc, axis=0)


def _projection(features, W, att2):
    n = features.shape[0]
    nb = n // _ROWS_B
    return pl.pallas_call(
        _proj_body,
        grid=(nb, 2),
        in_specs=[
            pl.BlockSpec((_ROWS_B, D), lambda i, h: (i, 0)),
            pl.BlockSpec((D, DH), lambda i, h: (0, h)),
            pl.BlockSpec((DH, 2), lambda i, h: (h, 0)),
        ],
        out_specs=[
            pl.BlockSpec((1, _ROWS_B, DH), lambda i, h: (h, i, 0)),
            pl.BlockSpec((_ROWS_B, 2), lambda i, h: (i, 0)),
            pl.BlockSpec((1, 1, 2), lambda i, h: (i, 0, 0)),
        ],
        out_shape=[
            jax.ShapeDtypeStruct((2, n, DH), jnp.float32),
            jax.ShapeDtypeStruct((n, 2), jnp.float32),
            jax.ShapeDtypeStruct((nb, 1, 2), jnp.float32),
        ],
    )(features, W, att2)


_VR_A = EW_A // 16
_RED = NP // NS


def _edge_logits_body(src_hbm, dst_hbm, asrc_hbm, adst_hbm, mvec_hbm,
                      ex_hbm, den_hbm,
                      as_v, ad_v, mv_v, src_v, dst_v, ex_v, ldenom,
                      shared_den, racc_v, rtmp_v):
    c = lax.axis_index("c")
    s = lax.axis_index("s")
    wid = c * NS + s
    base = wid * EW_A

    pltpu.sync_copy(asrc_hbm, as_v)
    pltpu.sync_copy(adst_hbm, ad_v)
    pltpu.sync_copy(mvec_hbm, mv_v)
    pltpu.sync_copy(src_hbm.at[pl.ds(base, EW_A)], src_v)
    pltpu.sync_copy(dst_hbm.at[pl.ds(base, EW_A)], dst_v)

    zeros16 = jnp.zeros((16,), jnp.float32)

    def zbody(i, _):
        ldenom[pl.ds(i * 16, 16)] = zeros16
        return 0

    lax.fori_loop(0, NP // 16, zbody, 0)

    mv = mv_v[...]

    def ebody(t, _):
        sl = pl.ds(t * 16, 16)
        si = src_v[sl]
        di = dst_v[sl]
        a_s = plsc.load_gather(as_v, [si])
        a_d = plsc.load_gather(ad_v, [di])
        sm = a_s + a_d
        e = jnp.where(sm > 0, sm, 0.2 * sm)
        ex = jnp.exp(e - mv)
        ex_v[sl] = ex
        plsc.addupdate_scatter(ldenom, [di], ex)
        return 0

    lax.fori_loop(0, _VR_A, ebody, 0)

    pltpu.sync_copy(ex_v, ex_hbm.at[pl.ds(base, EW_A)])
    pltpu.sync_copy(ldenom, shared_den.at[s])
    plsc.subcore_barrier()

    rbase = s * _RED
    pltpu.sync_copy(shared_den.at[0, pl.ds(rbase, _RED)], racc_v)

    def rbody(k, _):
        pltpu.sync_copy(shared_den.at[k, pl.ds(rbase, _RED)], rtmp_v)

        def abody(q, _):
            sl = pl.ds(q * 16, 16)
            racc_v[sl] = racc_v[sl] + rtmp_v[sl]
            return 0

        lax.fori_loop(0, _RED // 16, abody, 0)
        return 0

    lax.fori_loop(1, NS, rbody, 0)
    pltpu.sync_copy(racc_v, den_hbm.at[c, pl.ds(rbase, _RED)])


def _edge_logits(src_pad, dst_pad, asrc_pad, adst_pad, mvec):
    mesh = plsc.VectorSubcoreMesh(
        core_axis_name="c", subcore_axis_name="s", num_cores=NC,
        num_subcores=NS)
    f = pl.kernel(
        _edge_logits_body,
        out_type=[
            jax.ShapeDtypeStruct((E_PAD,), jnp.float32),
            jax.ShapeDtypeStruct((NC, NP), jnp.float32),
        ],
        mesh=mesh,
        compiler_params=pltpu.CompilerParams(needs_layout_passes=False),
        scratch_types=[
            pltpu.VMEM((NP,), jnp.float32),
            pltpu.VMEM((NP,), jnp.float32),
            pltpu.VMEM((16,), jnp.float32),
            pltpu.VMEM((EW_A,), jnp.int32),
            pltpu.VMEM((EW_A,), jnp.int32),
            pltpu.VMEM((EW_A,), jnp.float32),
            pltpu.VMEM((NP,), jnp.float32),
            pltpu.VMEM_SHARED((NS, NP), jnp.float32),
            pltpu.VMEM((_RED,), jnp.float32),
            pltpu.VMEM((_RED,), jnp.float32),
        ],
    )
    return f(src_pad, dst_pad, asrc_pad, adst_pad, mvec)


def kernel(features, edge_index, pos_edge_index, neg_edge_index,
           neg_sample_size, W, att_src, att_dst, bias):
    n = features.shape[0]
    att2 = jnp.stack([att_src, att_dst], axis=1)
    h_pair, apair, amax = _projection(features, W, att2)

    mx = jnp.max(amax, axis=(0, 1))
    msum = mx[0] + mx[1]
    M = jnp.where(msum > 0, msum, 0.2 * msum)

    h = jnp.concatenate([h_pair[0], h_pair[1]], axis=1)

    src = edge_index[0]
    dst = edge_index[1]
    n_edges = src.shape[0]

    src_pad = jnp.concatenate(
        [src, jnp.zeros((E_PAD - n_edges,), jnp.int32)])
    dst_pad = jnp.concatenate(
        [dst, jnp.full((E_PAD - n_edges,), N_NODES, jnp.int32)])
    zpad = jnp.zeros((NP - n,), jnp.float32)
    asrc_pad = jnp.concatenate([apair[:, 0], zpad])
    adst_pad = jnp.concatenate([apair[:, 1], zpad])
    mvec = jnp.full((16,), M, jnp.float32)

    ex_pad, denom_p = _edge_logits(src_pad, dst_pad, asrc_pad, adst_pad, mvec)
    ex = ex_pad[:n_edges]
    denom = (denom_p[0] + denom_p[1])[:n]
    alpha = ex / (denom[dst] + 1e-16)
    msgs = alpha[:, None] * h[src]
    out = jax.ops.segment_sum(msgs, dst, num_segments=n) + bias
    emb = jnp.where(out > 0, out, jnp.exp(jnp.minimum(out, 0.0)) - 1.0)

    def score(eidx):
        return jnp.sum(emb[eidx[0]] * emb[eidx[1]], axis=1)

    pos_score = score(pos_edge_index)
    neg_score = score(neg_edge_index)
    pos_l = jax.nn.log_sigmoid(pos_score)
    neg_l = jax.nn.log_sigmoid(-neg_score)
    loss = -(jnp.sum(pos_l) + jnp.sum(neg_l)) / pos_score.shape[0]
    return emb, loss


# speedup vs baseline: 3.5471x; 2.1855x over previous
import functools

import jax
import jax.numpy as jnp
import numpy as np
from jax import lax
from jax.experimental import pallas as pl
from jax.experimental.pallas import tpu as pltpu
from jax.experimental.pallas import tpu_sc as plsc

N_NODES = 10000
D = 256
DH = 128
DQ = 64

NC = 2
NS = 16
NW = NC * NS

NP = 10240
E_PAD = 163840
EW_A = E_PAD // NW


_ROWS_B = 1000


def _proj_body(x_ref, w_ref, att_ref, h_ref, apair_ref, amax_ref):
    half = pl.program_id(1)
    hb = jnp.dot(x_ref[...], w_ref[...], preferred_element_type=jnp.float32)
    h_ref[0] = hb
    ap = jnp.dot(hb, att_ref[...], preferred_element_type=jnp.float32)

    @pl.when(half == 0)
    def _():
        apair_ref[...] = ap

    @pl.when(half == 1)
    def _():
        acc = apair_ref[...] + ap
        apair_ref[...] = acc
        amax_ref[0, 0, :] = jnp.max(acc, axis=0)


def _projection(features, W, att2):
    n = features.shape[0]
    nb = n // _ROWS_B
    return pl.pallas_call(
        _proj_body,
        grid=(nb, 2),
        in_specs=[
            pl.BlockSpec((_ROWS_B, D), lambda i, h: (i, 0)),
            pl.BlockSpec((D, DH), lambda i, h: (0, h)),
            pl.BlockSpec((DH, 2), lambda i, h: (h, 0)),
        ],
        out_specs=[
            pl.BlockSpec((1, _ROWS_B, DH), lambda i, h: (h, i, 0)),
            pl.BlockSpec((_ROWS_B, 2), lambda i, h: (i, 0)),
            pl.BlockSpec((1, 1, 2), lambda i, h: (i, 0, 0)),
        ],
        out_shape=[
            jax.ShapeDtypeStruct((2, n, DH), jnp.float32),
            jax.ShapeDtypeStruct((n, 2), jnp.float32),
            jax.ShapeDtypeStruct((nb, 1, 2), jnp.float32),
        ],
    )(features, W, att2)


_VR_A = EW_A // 16
NPH = NP // 2
_RED = NPH // 8


def _edge_logits_body(src_hbm, dst_hbm, asrc_hbm, adst_hbm, mvec_hbm,
                      ex_hbm, den_hbm,
                      as_v, ad_v, mv_v, src_v, dst_v, ex_v, ldenom,
                      shared_den, racc_v, rtmp_v):
    c = lax.axis_index("c")
    s = lax.axis_index("s")
    wid = c * NS + s
    base = wid * EW_A

    pltpu.sync_copy(asrc_hbm, as_v)
    pltpu.sync_copy(adst_hbm, ad_v)
    pltpu.sync_copy(mvec_hbm, mv_v)
    pltpu.sync_copy(src_hbm.at[pl.ds(base, EW_A)], src_v)
    pltpu.sync_copy(dst_hbm.at[pl.ds(base, EW_A)], dst_v)

    zeros16 = jnp.zeros((16,), jnp.float32)

    def zbody(i, _):
        ldenom[pl.ds(i * 16, 16)] = zeros16
        return 0

    lax.fori_loop(0, NP // 16, zbody, 0)

    mv = mv_v[...]

    def ebody(t, _):
        sl = pl.ds(t * 16, 16)
        si = src_v[sl]
        di = dst_v[sl]
        a_s = plsc.load_gather(as_v, [si])
        a_d = plsc.load_gather(ad_v, [di])
        sm = a_s + a_d
        e = jnp.where(sm > 0, sm, 0.2 * sm)
        ex = jnp.exp(e - mv)
        ex_v[sl] = ex
        plsc.addupdate_scatter(ldenom, [di], ex)
        return 0

    lax.fori_loop(0, _VR_A, ebody, 0)

    pltpu.sync_copy(ex_v, ex_hbm.at[pl.ds(base, EW_A)])

    for half in range(2):
        pltpu.sync_copy(ldenom.at[pl.ds(half * NPH, NPH)],
                        shared_den.at[pl.ds(s * NPH, NPH)])
        plsc.subcore_barrier()
        active = (s < 8) if half == 0 else (s >= 8)

        @pl.when(active)
        def _():
            rbase = (s - 8 * half) * _RED
            pltpu.sync_copy(shared_den.at[pl.ds(rbase, _RED)], racc_v)

            def rbody(k, _):
                pltpu.sync_copy(
                    shared_den.at[pl.ds(k * NPH + rbase, _RED)], rtmp_v)

                def abody(q, _):
                    sl = pl.ds(q * 16, 16)
                    racc_v[sl] = racc_v[sl] + rtmp_v[sl]
                    return 0

                lax.fori_loop(0, _RED // 16, abody, 0)
                return 0

            lax.fori_loop(1, NS, rbody, 0)
            pltpu.sync_copy(racc_v,
                            den_hbm.at[c, pl.ds(half * NPH + rbase, _RED)])

        plsc.subcore_barrier()


def _edge_logits(src_pad, dst_pad, asrc_pad, adst_pad, mvec):
    mesh = plsc.VectorSubcoreMesh(
        core_axis_name="c", subcore_axis_name="s", num_cores=NC,
        num_subcores=NS)
    f = pl.kernel(
        _edge_logits_body,
        out_type=[
            jax.ShapeDtypeStruct((E_PAD,), jnp.float32),
            jax.ShapeDtypeStruct((NC, NP), jnp.float32),
        ],
        mesh=mesh,
        compiler_params=pltpu.CompilerParams(needs_layout_passes=False),
        scratch_types=[
            pltpu.VMEM((NP,), jnp.float32),
            pltpu.VMEM((NP,), jnp.float32),
            pltpu.VMEM((16,), jnp.float32),
            pltpu.VMEM((EW_A,), jnp.int32),
            pltpu.VMEM((EW_A,), jnp.int32),
            pltpu.VMEM((EW_A,), jnp.float32),
            pltpu.VMEM((NP,), jnp.float32),
            pltpu.VMEM_SHARED((NS * NPH,), jnp.float32),
            pltpu.VMEM((_RED,), jnp.float32),
            pltpu.VMEM((_RED,), jnp.float32),
        ],
    )
    return f(src_pad, dst_pad, asrc_pad, adst_pad, mvec)


_EW_B = E_PAD // NS
_NG_B = _EW_B // 128
NPASS = 4
HR = NP // NPASS
_ACC_R = 2688
_IR_B = _ACC_R // NS
_ER_B = HR // NS


def _msg_body(src_hbm, dst_hbm, ex_hbm, den_hbm, h_hbm, bias_hbm,
              emb_hbm,
              d0_v, inv_v, src_v, dst_v, ex_v, alpha_v,
              ra, rb, dst_g, bias_v, acc_sh, sem_a, sem_b):
    c = lax.axis_index("c")
    s = lax.axis_index("s")

    pltpu.sync_copy(den_hbm.at[0], d0_v)
    pltpu.sync_copy(den_hbm.at[1], inv_v)

    def dbody(i, _):
        sl = pl.ds(i * 16, 16)
        inv_v[sl] = 1.0 / (d0_v[sl] + inv_v[sl] + 1e-16)
        return 0

    lax.fori_loop(0, NP // 16, dbody, 0)

    base = s * _EW_B
    pltpu.sync_copy(src_hbm.at[pl.ds(base, _EW_B)], src_v)
    pltpu.sync_copy(dst_hbm.at[pl.ds(base, _EW_B)], dst_v)
    pltpu.sync_copy(ex_hbm.at[pl.ds(base, _EW_B)], ex_v)

    def abody(t, _):
        sl = pl.ds(t * 16, 16)
        di = dst_v[sl]
        iv = plsc.load_gather(inv_v, [di])
        alpha_v[sl] = ex_v[sl] * iv
        return 0

    lax.fori_loop(0, _EW_B // 16, abody, 0)

    hc = h_hbm.at[c]
    pltpu.sync_copy(bias_hbm.at[c], bias_v)
    bvs = [bias_v[pl.ds(16 * k, 16)] for k in range(8)]
    trash = HR + s

    def issue(g, r_buf, sem):
        pltpu.async_copy(hc.at[src_v.at[pl.ds(g * 128, 128)]], r_buf, sem)

    for r in range(NPASS):
        def ibody(row, _):
            for k in range(8):
                ra[row, pl.ds(16 * k, 16)] = bvs[k]
            return 0

        lax.fori_loop(0, 128, ibody, 0)
        i0 = s * _IR_B
        pltpu.sync_copy(ra, acc_sh.at[pl.ds(i0, 128)])
        pltpu.sync_copy(ra.at[pl.ds(0, _IR_B - 128)],
                        acc_sh.at[pl.ds(i0 + 128, _IR_B - 128)])
        plsc.subcore_barrier()

        issue(0, ra, sem_a)
        issue(1, rb, sem_b)

        def process(g, r_buf):
            gbase = g * 128
            for k in range(8):
                dl = dst_v[pl.ds(gbase + 16 * k, 16)] - (r * HR)
                ok = (dl >= 0) & (dl < HR)
                dst_g[pl.ds(16 * k, 16)] = jnp.where(ok, dl, trash)

            def pbody(j, _):
                av = plsc.load_gather(
                    alpha_v, [jnp.full((16,), gbase + j, jnp.int32)])
                for k in range(8):
                    sl = pl.ds(16 * k, 16)
                    r_buf[j, sl] = r_buf[j, sl] * av
                return 0

            lax.fori_loop(0, 128, pbody, 0)
            pltpu.sync_copy(r_buf, acc_sh.at[dst_g], add=True)

        def outer(t, _):
            for b in range(2):
                r_buf, sem = (ra, sem_a) if b == 0 else (rb, sem_b)
                g = t * 2 + b
                pltpu.make_async_copy(
                    hc.at[pl.ds(0, 128)], r_buf, sem).wait()
                process(g, r_buf)

                @pl.when(g + 2 < _NG_B)
                def _():
                    issue(g + 2, r_buf, sem)

            return 0

        lax.fori_loop(0, _NG_B // 2, outer, 0)
        plsc.subcore_barrier()

        for t, chunk in ((0, 128), (128, _ER_B - 128)):
            r0 = s * _ER_B + t
            pltpu.sync_copy(acc_sh.at[pl.ds(r0, chunk)],
                            ra.at[pl.ds(0, chunk)])

            def erow(row, _):
                for k in range(8):
                    sl = pl.ds(16 * k, 16)
                    x = ra[row, sl]
                    ra[row, sl] = jnp.where(
                        x > 0, x, jnp.exp(jnp.minimum(x, 0.0)) - 1.0)
                return 0

            lax.fori_loop(0, chunk, erow, 0)
            pltpu.sync_copy(ra.at[pl.ds(0, chunk)],
                            emb_hbm.at[c].at[pl.ds(r * HR + r0, chunk)])
        plsc.subcore_barrier()


def _message_pass(src_pad, dst_pad, ex_pad, denom_p, h_pair, bias2):
    mesh = plsc.VectorSubcoreMesh(
        core_axis_name="c", subcore_axis_name="s", num_cores=NC,
        num_subcores=NS)
    f = pl.kernel(
        _msg_body,
        out_type=jax.ShapeDtypeStruct((NC, NP, DH), jnp.float32),
        mesh=mesh,
        compiler_params=pltpu.CompilerParams(needs_layout_passes=False),
        scratch_types=[
            pltpu.VMEM((NP,), jnp.float32),
            pltpu.VMEM((NP,), jnp.float32),
            pltpu.VMEM((_EW_B,), jnp.int32),
            pltpu.VMEM((_EW_B,), jnp.int32),
            pltpu.VMEM((_EW_B,), jnp.float32),
            pltpu.VMEM((_EW_B,), jnp.float32),
            pltpu.VMEM((128, DH), jnp.float32),
            pltpu.VMEM((128, DH), jnp.float32),
            pltpu.VMEM((128,), jnp.int32),
            pltpu.VMEM((DH,), jnp.float32),
            pltpu.VMEM_SHARED((_ACC_R, DH), jnp.float32),
            pltpu.SemaphoreType.DMA,
            pltpu.SemaphoreType.DMA,
        ],
    )
    return f(src_pad, dst_pad, ex_pad, denom_p, h_pair, bias2)


SE = 278528
_EW_C = SE // NS
_NG_C = _EW_C // 128


def _score_body(hh_hbm, tt_hbm, emb_hbm,
                pp_hbm,
                hh_v, tt_v, ha, ta, hb, tb, pout, sem_a, sem_b):
    c = lax.axis_index("c")
    s = lax.axis_index("s")
    base = s * _EW_C

    pltpu.sync_copy(hh_hbm.at[pl.ds(base, _EW_C)], hh_v)
    pltpu.sync_copy(tt_hbm.at[pl.ds(base, _EW_C)], tt_v)

    ec = emb_hbm.at[c]

    def issue(g, hbuf, tbuf, sem):
        pltpu.async_copy(ec.at[hh_v.at[pl.ds(g * 128, 128)]], hbuf, sem)
        pltpu.async_copy(ec.at[tt_v.at[pl.ds(g * 128, 128)]], tbuf, sem)

    issue(0, ha, ta, sem_a)
    issue(1, hb, tb, sem_b)

    def process(g, hbuf, tbuf):
        def pbody(j, _):
            acc = hbuf[j, pl.ds(0, 16)] * tbuf[j, pl.ds(0, 16)]
            for k in range(1, 8):
                sl = pl.ds(16 * k, 16)
                acc = acc + hbuf[j, sl] * tbuf[j, sl]
            pout[j, :] = acc
            return 0

        lax.fori_loop(0, 128, pbody, 0)
        pltpu.sync_copy(pout, pp_hbm.at[c].at[pl.ds(base + g * 128, 128)])

    def outer(t, _):
        for b in range(2):
            hbuf, tbuf, sem = (ha, ta, sem_a) if b == 0 else (hb, tb, sem_b)
            g = t * 2 + b
            pltpu.make_async_copy(ec.at[pl.ds(0, 128)], hbuf, sem).wait()
            pltpu.make_async_copy(ec.at[pl.ds(0, 128)], tbuf, sem).wait()
            process(g, hbuf, tbuf)

            @pl.when(g + 2 < _NG_C)
            def _():
                issue(g + 2, hbuf, tbuf, sem)

        return 0

    lax.fori_loop(0, _NG_C // 2, outer, 0)


def _score(heads, tails, emb_pair):
    mesh = plsc.VectorSubcoreMesh(
        core_axis_name="c", subcore_axis_name="s", num_cores=NC,
        num_subcores=NS)
    f = pl.kernel(
        _score_body,
        out_type=jax.ShapeDtypeStruct((NC, SE, 16), jnp.float32),
        mesh=mesh,
        compiler_params=pltpu.CompilerParams(needs_layout_passes=False),
        scratch_types=[
            pltpu.VMEM((_EW_C,), jnp.int32),
            pltpu.VMEM((_EW_C,), jnp.int32),
            pltpu.VMEM((128, DH), jnp.float32),
            pltpu.VMEM((128, DH), jnp.float32),
            pltpu.VMEM((128, DH), jnp.float32),
            pltpu.VMEM((128, DH), jnp.float32),
            pltpu.VMEM((128, 16), jnp.float32),
            pltpu.SemaphoreType.DMA,
            pltpu.SemaphoreType.DMA,
        ],
    )
    return f(heads, tails, emb_pair)


_LROWS = 2048


def _loss_body(pp_ref, g_ref, out_ref):
    pid = pl.program_id(0)
    x = pp_ref[0] + pp_ref[1]
    s = jnp.dot(x, g_ref[...], preferred_element_type=jnp.float32)
    s = jnp.where(pid == 0, s, -s)
    ls = jnp.minimum(s, 0.0) - jnp.log1p(jnp.exp(-jnp.abs(s)))
    bsum = jnp.sum(ls).reshape(1, 1)
    nblk = pl.num_programs(0)
    acc = jnp.where(pid == 0, jnp.zeros((1, 1), jnp.float32),
                    out_ref[...]) + bsum
    out_ref[...] = jnp.where(pid == nblk - 1, -acc / 16384.0, acc)


def _loss(pp3):
    nrows = pp3.shape[1]
    grid = nrows // _LROWS
    gmat = np.zeros((128, 8), np.float32)
    for l in range(128):
        gmat[l, l // 16] = 1.0
    return pl.pallas_call(
        _loss_body,
        grid=(grid,),
        in_specs=[
            pl.BlockSpec((NC, _LROWS, 128), lambda i: (0, i, 0)),
            pl.BlockSpec((128, 8), lambda i: (0, 0)),
        ],
        out_specs=pl.BlockSpec((1, 1), lambda i: (0, 0)),
        out_shape=jax.ShapeDtypeStruct((1, 1), jnp.float32),
    )(pp3, jnp.asarray(gmat))


def kernel(features, edge_index, pos_edge_index, neg_edge_index,
           neg_sample_size, W, att_src, att_dst, bias):
    n = features.shape[0]
    att2 = jnp.stack([att_src, att_dst], axis=1)
    h_pair, apair, amax = _projection(features, W, att2)

    mx = jnp.max(amax, axis=(0, 1))
    msum = mx[0] + mx[1]
    M = jnp.where(msum > 0, msum, 0.2 * msum)

    src = edge_index[0]
    dst = edge_index[1]
    n_edges = src.shape[0]

    src_pad = jnp.concatenate(
        [src, jnp.zeros((E_PAD - n_edges,), jnp.int32)])
    dst_pad = jnp.concatenate(
        [dst, jnp.full((E_PAD - n_edges,), N_NODES, jnp.int32)])
    zpad = jnp.zeros((NP - n,), jnp.float32)
    asrc_pad = jnp.concatenate([apair[:, 0], zpad])
    adst_pad = jnp.concatenate([apair[:, 1], zpad])
    mvec = jnp.full((16,), M, jnp.float32)

    ex_pad, denom_p = _edge_logits(src_pad, dst_pad, asrc_pad, adst_pad, mvec)

    bias2 = bias.reshape(NC, DH)
    emb_pair = _message_pass(src_pad, dst_pad, ex_pad, denom_p, h_pair, bias2)
    emb = jnp.concatenate([emb_pair[0, :n], emb_pair[1, :n]], axis=1)

    sedges = jnp.concatenate([pos_edge_index, neg_edge_index], axis=1)
    pp = _score(sedges[0], sedges[1], emb_pair)
    loss = _loss(pp.reshape(NC, SE * 16 // 128, 128))[0, 0]
    return emb, loss


# speedup vs baseline: 4.3608x; 1.2294x over previous
import functools

import jax
import jax.numpy as jnp
import numpy as np
from jax import lax
from jax.experimental import pallas as pl
from jax.experimental.pallas import tpu as pltpu
from jax.experimental.pallas import tpu_sc as plsc

N_NODES = 10000
D = 256
DH = 128
DQ = 64

NC = 2
NS = 16
NW = NC * NS

NP = 10240
E_PAD = 163840
EW_A = E_PAD // NW


_ROWS_B = 1000


def _proj_body(x_ref, w_ref, att_ref, h_ref, apair_ref, amax_ref):
    half = pl.program_id(1)
    hb = jnp.dot(x_ref[...], w_ref[...], preferred_element_type=jnp.float32)
    h_ref[0] = hb
    ap = jnp.dot(hb, att_ref[...], preferred_element_type=jnp.float32)

    @pl.when(half == 0)
    def _():
        apair_ref[...] = ap

    @pl.when(half == 1)
    def _():
        acc = apair_ref[...] + ap
        apair_ref[...] = acc
        amax_ref[0, 0, :] = jnp.max(acc, axis=0)


def _projection(features, W, att2):
    n = features.shape[0]
    nb = n // _ROWS_B
    return pl.pallas_call(
        _proj_body,
        grid=(nb, 2),
        in_specs=[
            pl.BlockSpec((_ROWS_B, D), lambda i, h: (i, 0)),
            pl.BlockSpec((D, DH), lambda i, h: (0, h)),
            pl.BlockSpec((DH, 2), lambda i, h: (h, 0)),
        ],
        out_specs=[
            pl.BlockSpec((1, _ROWS_B, DH), lambda i, h: (h, i, 0)),
            pl.BlockSpec((_ROWS_B, 2), lambda i, h: (i, 0)),
            pl.BlockSpec((1, 1, 2), lambda i, h: (i, 0, 0)),
        ],
        out_shape=[
            jax.ShapeDtypeStruct((2, n, DH), jnp.float32),
            jax.ShapeDtypeStruct((n, 2), jnp.float32),
            jax.ShapeDtypeStruct((nb, 1, 2), jnp.float32),
        ],
    )(features, W, att2)


_VR_A = EW_A // 16
NPH = NP // 2
_RED = NPH // 8


def _edge_logits_body(src_hbm, dst_hbm, asrc_hbm, adst_hbm, mvec_hbm,
                      ex_hbm, den_hbm,
                      as_v, ad_v, mv_v, src_v, dst_v, ex_v, ldenom,
                      shared_den, racc_v, rtmp_v):
    c = lax.axis_index("c")
    s = lax.axis_index("s")
    wid = c * NS + s
    base = wid * EW_A

    pltpu.sync_copy(asrc_hbm, as_v)
    pltpu.sync_copy(adst_hbm, ad_v)
    pltpu.sync_copy(mvec_hbm, mv_v)
    pltpu.sync_copy(src_hbm.at[pl.ds(base, EW_A)], src_v)
    pltpu.sync_copy(dst_hbm.at[pl.ds(base, EW_A)], dst_v)

    zeros16 = jnp.zeros((16,), jnp.float32)

    def zbody(i, _):
        ldenom[pl.ds(i * 16, 16)] = zeros16
        return 0

    lax.fori_loop(0, NP // 16, zbody, 0)

    mv = mv_v[...]

    def ebody(t, _):
        sl = pl.ds(t * 16, 16)
        si = src_v[sl]
        di = dst_v[sl]
        a_s = plsc.load_gather(as_v, [si])
        a_d = plsc.load_gather(ad_v, [di])
        sm = a_s + a_d
        e = jnp.where(sm > 0, sm, 0.2 * sm)
        ex = jnp.exp(e - mv)
        ex_v[sl] = ex
        plsc.addupdate_scatter(ldenom, [di], ex)
        return 0

    lax.fori_loop(0, _VR_A, ebody, 0)

    pltpu.sync_copy(ex_v, ex_hbm.at[pl.ds(base, EW_A)])

    for half in range(2):
        pltpu.sync_copy(ldenom.at[pl.ds(half * NPH, NPH)],
                        shared_den.at[pl.ds(s * NPH, NPH)])
        plsc.subcore_barrier()
        active = (s < 8) if half == 0 else (s >= 8)

        @pl.when(active)
        def _():
            rbase = (s - 8 * half) * _RED
            pltpu.sync_copy(shared_den.at[pl.ds(rbase, _RED)], racc_v)

            def rbody(k, _):
                pltpu.sync_copy(
                    shared_den.at[pl.ds(k * NPH + rbase, _RED)], rtmp_v)

                def abody(q, _):
                    sl = pl.ds(q * 16, 16)
                    racc_v[sl] = racc_v[sl] + rtmp_v[sl]
                    return 0

                lax.fori_loop(0, _RED // 16, abody, 0)
                return 0

            lax.fori_loop(1, NS, rbody, 0)
            pltpu.sync_copy(racc_v,
                            den_hbm.at[c, pl.ds(half * NPH + rbase, _RED)])

        plsc.subcore_barrier()


def _edge_logits(src_pad, dst_pad, asrc_pad, adst_pad, mvec):
    mesh = plsc.VectorSubcoreMesh(
        core_axis_name="c", subcore_axis_name="s", num_cores=NC,
        num_subcores=NS)
    f = pl.kernel(
        _edge_logits_body,
        out_type=[
            jax.ShapeDtypeStruct((E_PAD,), jnp.float32),
            jax.ShapeDtypeStruct((NC, NP), jnp.float32),
        ],
        mesh=mesh,
        compiler_params=pltpu.CompilerParams(needs_layout_passes=False),
        scratch_types=[
            pltpu.VMEM((NP,), jnp.float32),
            pltpu.VMEM((NP,), jnp.float32),
            pltpu.VMEM((16,), jnp.float32),
            pltpu.VMEM((EW_A,), jnp.int32),
            pltpu.VMEM((EW_A,), jnp.int32),
            pltpu.VMEM((EW_A,), jnp.float32),
            pltpu.VMEM((NP,), jnp.float32),
            pltpu.VMEM_SHARED((NS * NPH,), jnp.float32),
            pltpu.VMEM((_RED,), jnp.float32),
            pltpu.VMEM((_RED,), jnp.float32),
        ],
    )
    return f(src_pad, dst_pad, asrc_pad, adst_pad, mvec)


_EW_B = E_PAD // NS
_NG_B = _EW_B // 128
NPASS = 3
HR = 3456
NPB = NPASS * HR
_ACC_R = 3584
_IR_B = _ACC_R // NS
_ER_B = HR // NS


def _msg_body(src_hbm, dst_hbm, ex_hbm, den_hbm, h_hbm, bias_hbm,
              emb_hbm,
              inv_v, src_v, dst_v, ex_v, alpha_v,
              ra, rb, dst_g, bias_v, acc_sh,
              sem_a, sem_b):
    c = lax.axis_index("c")
    s = lax.axis_index("s")

    pltpu.sync_copy(den_hbm.at[0], inv_v)
    pltpu.sync_copy(den_hbm.at[1], alpha_v)

    def dbody(i, _):
        sl = pl.ds(i * 16, 16)
        inv_v[sl] = 1.0 / (inv_v[sl] + alpha_v[sl] + 1e-16)
        return 0

    lax.fori_loop(0, NP // 16, dbody, 0)

    base = s * _EW_B
    pltpu.sync_copy(src_hbm.at[pl.ds(base, _EW_B)], src_v)
    pltpu.sync_copy(dst_hbm.at[pl.ds(base, _EW_B)], dst_v)
    pltpu.sync_copy(ex_hbm.at[pl.ds(base, _EW_B)], ex_v)

    def abody(t, _):
        sl = pl.ds(t * 16, 16)
        di = dst_v[sl]
        iv = plsc.load_gather(inv_v, [di])
        alpha_v[sl] = ex_v[sl] * iv
        return 0

    lax.fori_loop(0, _EW_B // 16, abody, 0)

    hc = h_hbm.at[c]
    pltpu.sync_copy(bias_hbm.at[c], bias_v)
    bvs = [bias_v[pl.ds(16 * k, 16)] for k in range(8)]
    trash = HR + s

    def issue(g, r_buf, sem):
        pltpu.async_copy(hc.at[src_v.at[pl.ds(g * 128, 128)]], r_buf, sem)

    for r in range(NPASS):
        def ibody(row, _):
            for k in range(8):
                ra[row, pl.ds(16 * k, 16)] = bvs[k]
            return 0

        lax.fori_loop(0, 128, ibody, 0)
        i0 = s * _IR_B
        pltpu.sync_copy(ra, acc_sh.at[pl.ds(i0, 128)])
        pltpu.sync_copy(ra.at[pl.ds(0, _IR_B - 128)],
                        acc_sh.at[pl.ds(i0 + 128, _IR_B - 128)])
        plsc.subcore_barrier()

        issue(0, ra, sem_a)
        issue(1, rb, sem_b)

        def process(g, r_buf):
            gbase = g * 128
            for k in range(8):
                dl = dst_v[pl.ds(gbase + 16 * k, 16)] - (r * HR)
                ok = (dl >= 0) & (dl < HR)
                dst_g[pl.ds(16 * k, 16)] = jnp.where(ok, dl, trash)

            def pbody(j, _):
                av = plsc.load_gather(
                    alpha_v, [jnp.full((16,), gbase + j, jnp.int32)])
                for k in range(8):
                    sl = pl.ds(16 * k, 16)
                    r_buf[j, sl] = r_buf[j, sl] * av
                return 0

            lax.fori_loop(0, 128, pbody, 0)
            pltpu.sync_copy(r_buf, acc_sh.at[dst_g], add=True)

        def outer(t, _):
            for b in range(2):
                r_buf, sem = (ra, sem_a) if b == 0 else (rb, sem_b)
                g = t * 2 + b
                pltpu.make_async_copy(
                    hc.at[pl.ds(0, 128)], r_buf, sem).wait()
                process(g, r_buf)

                @pl.when(g + 2 < _NG_B)
                def _():
                    issue(g + 2, r_buf, sem)

            return 0

        lax.fori_loop(0, _NG_B // 2, outer, 0)
        plsc.subcore_barrier()

        for t, chunk in ((0, 128), (128, _ER_B - 128)):
            r0 = s * _ER_B + t
            pltpu.sync_copy(acc_sh.at[pl.ds(r0, chunk)],
                            ra.at[pl.ds(0, chunk)])

            def erow(row, _):
                for k in range(8):
                    sl = pl.ds(16 * k, 16)
                    x = ra[row, sl]
                    ra[row, sl] = jnp.where(
                        x > 0, x, jnp.exp(jnp.minimum(x, 0.0)) - 1.0)
                return 0

            lax.fori_loop(0, chunk, erow, 0)
            pltpu.sync_copy(ra.at[pl.ds(0, chunk)],
                            emb_hbm.at[c].at[pl.ds(r * HR + r0, chunk)])
        plsc.subcore_barrier()


def _message_pass(src_pad, dst_pad, ex_pad, denom_p, h_pair, bias2):
    mesh = plsc.VectorSubcoreMesh(
        core_axis_name="c", subcore_axis_name="s", num_cores=NC,
        num_subcores=NS)
    f = pl.kernel(
        _msg_body,
        out_type=jax.ShapeDtypeStruct((NC, NPB, DH), jnp.float32),
        mesh=mesh,
        compiler_params=pltpu.CompilerParams(needs_layout_passes=False),
        scratch_types=[
            pltpu.VMEM((NP,), jnp.float32),
            pltpu.VMEM((_EW_B,), jnp.int32),
            pltpu.VMEM((_EW_B,), jnp.int32),
            pltpu.VMEM((_EW_B,), jnp.float32),
            pltpu.VMEM((_EW_B,), jnp.float32),
            pltpu.VMEM((128, DH), jnp.float32),
            pltpu.VMEM((128, DH), jnp.float32),
            pltpu.VMEM((128,), jnp.int32),
            pltpu.VMEM((DH,), jnp.float32),
            pltpu.VMEM_SHARED((_ACC_R, DH), jnp.float32),
            pltpu.SemaphoreType.DMA,
            pltpu.SemaphoreType.DMA,
        ],
    )
    return f(src_pad, dst_pad, ex_pad, denom_p, h_pair, bias2)


SE = 278528
_EW_C = SE // NS
_NG_C = _EW_C // 128


def _score_body(hh_hbm, tt_hbm, emb_hbm,
                pp_hbm,
                hh_v, tt_v, ha, ta, hb, tb, pout, sem_a, sem_b):
    c = lax.axis_index("c")
    s = lax.axis_index("s")
    base = s * _EW_C

    pltpu.sync_copy(hh_hbm.at[pl.ds(base, _EW_C)], hh_v)
    pltpu.sync_copy(tt_hbm.at[pl.ds(base, _EW_C)], tt_v)

    ec = emb_hbm.at[c]

    def issue(g, hbuf, tbuf, sem):
        pltpu.async_copy(ec.at[hh_v.at[pl.ds(g * 128, 128)]], hbuf, sem)
        pltpu.async_copy(ec.at[tt_v.at[pl.ds(g * 128, 128)]], tbuf, sem)

    issue(0, ha, ta, sem_a)
    issue(1, hb, tb, sem_b)

    def process(g, hbuf, tbuf):
        def pbody(j, _):
            acc = hbuf[j, pl.ds(0, 16)] * tbuf[j, pl.ds(0, 16)]
            for k in range(1, 8):
                sl = pl.ds(16 * k, 16)
                acc = acc + hbuf[j, sl] * tbuf[j, sl]
            pout[j, :] = acc
            return 0

        lax.fori_loop(0, 128, pbody, 0)
        pltpu.sync_copy(pout, pp_hbm.at[c].at[pl.ds(base + g * 128, 128)])

    def outer(t, _):
        for b in range(2):
            hbuf, tbuf, sem = (ha, ta, sem_a) if b == 0 else (hb, tb, sem_b)
            g = t * 2 + b
            pltpu.make_async_copy(ec.at[pl.ds(0, 128)], hbuf, sem).wait()
            pltpu.make_async_copy(ec.at[pl.ds(0, 128)], tbuf, sem).wait()
            process(g, hbuf, tbuf)

            @pl.when(g + 2 < _NG_C)
            def _():
                issue(g + 2, hbuf, tbuf, sem)

        return 0

    lax.fori_loop(0, _NG_C // 2, outer, 0)


def _score(heads, tails, emb_pair):
    mesh = plsc.VectorSubcoreMesh(
        core_axis_name="c", subcore_axis_name="s", num_cores=NC,
        num_subcores=NS)
    f = pl.kernel(
        _score_body,
        out_type=jax.ShapeDtypeStruct((NC, SE, 16), jnp.float32),
        mesh=mesh,
        compiler_params=pltpu.CompilerParams(needs_layout_passes=False),
        scratch_types=[
            pltpu.VMEM((_EW_C,), jnp.int32),
            pltpu.VMEM((_EW_C,), jnp.int32),
            pltpu.VMEM((128, DH), jnp.float32),
            pltpu.VMEM((128, DH), jnp.float32),
            pltpu.VMEM((128, DH), jnp.float32),
            pltpu.VMEM((128, DH), jnp.float32),
            pltpu.VMEM((128, 16), jnp.float32),
            pltpu.SemaphoreType.DMA,
            pltpu.SemaphoreType.DMA,
        ],
    )
    return f(heads, tails, emb_pair)


_LROWS = 2048


def _loss_body(pp_ref, g_ref, out_ref):
    pid = pl.program_id(0)
    x = pp_ref[0] + pp_ref[1]
    s = jnp.dot(x, g_ref[...], preferred_element_type=jnp.float32)
    s = jnp.where(pid == 0, s, -s)
    ls = jnp.minimum(s, 0.0) - jnp.log1p(jnp.exp(-jnp.abs(s)))
    bsum = jnp.sum(ls).reshape(1, 1)
    nblk = pl.num_programs(0)
    acc = jnp.where(pid == 0, jnp.zeros((1, 1), jnp.float32),
                    out_ref[...]) + bsum
    out_ref[...] = jnp.where(pid == nblk - 1, -acc / 16384.0, acc)


def _loss(pp3):
    nrows = pp3.shape[1]
    grid = nrows // _LROWS
    gmat = np.zeros((128, 8), np.float32)
    for l in range(128):
        gmat[l, l // 16] = 1.0
    return pl.pallas_call(
        _loss_body,
        grid=(grid,),
        in_specs=[
            pl.BlockSpec((NC, _LROWS, 128), lambda i: (0, i, 0)),
            pl.BlockSpec((128, 8), lambda i: (0, 0)),
        ],
        out_specs=pl.BlockSpec((1, 1), lambda i: (0, 0)),
        out_shape=jax.ShapeDtypeStruct((1, 1), jnp.float32),
    )(pp3, jnp.asarray(gmat))


def kernel(features, edge_index, pos_edge_index, neg_edge_index,
           neg_sample_size, W, att_src, att_dst, bias):
    n = features.shape[0]
    att2 = jnp.stack([att_src, att_dst], axis=1)
    h_pair, apair, amax = _projection(features, W, att2)

    mx = jnp.max(amax, axis=(0, 1))
    msum = mx[0] + mx[1]
    M = jnp.where(msum > 0, msum, 0.2 * msum)

    src = edge_index[0]
    dst = edge_index[1]
    n_edges = src.shape[0]

    src_pad = jnp.concatenate(
        [src, jnp.zeros((E_PAD - n_edges,), jnp.int32)])
    dst_pad = jnp.concatenate(
        [dst, jnp.full((E_PAD - n_edges,), N_NODES, jnp.int32)])
    zpad = jnp.zeros((NP - n,), jnp.float32)
    asrc_pad = jnp.concatenate([apair[:, 0], zpad])
    adst_pad = jnp.concatenate([apair[:, 1], zpad])
    mvec = jnp.full((16,), M, jnp.float32)

    ex_pad, denom_p = _edge_logits(src_pad, dst_pad, asrc_pad, adst_pad, mvec)

    bias2 = bias.reshape(NC, DH)
    emb_pair = _message_pass(src_pad, dst_pad, ex_pad, denom_p, h_pair, bias2)
    emb = jnp.concatenate([emb_pair[0, :n], emb_pair[1, :n]], axis=1)

    sedges = jnp.concatenate([pos_edge_index, neg_edge_index], axis=1)
    pp = _score(sedges[0], sedges[1], emb_pair)
    loss = _loss(pp.reshape(NC, SE * 16 // 128, 128))[0, 0]
    return emb, loss


# speedup vs baseline: 4.9235x; 1.1290x over previous
import functools

import jax
import jax.numpy as jnp
import numpy as np
from jax import lax
from jax.experimental import pallas as pl
from jax.experimental.pallas import tpu as pltpu
from jax.experimental.pallas import tpu_sc as plsc

N_NODES = 10000
D = 256
DH = 128
DQ = 64

NC = 2
NS = 16
NW = NC * NS

NP = 10240
E_PAD = 163840
EW_A = E_PAD // NW
NPASS = 3
HR = 3456
NPB = NPASS * HR
PAD_DST = 16384


_ROWS_B = 1000


def _proj_body(x_ref, w_ref, att_ref, h_ref, apair_ref, amax_ref):
    half = pl.program_id(1)
    hb = jnp.dot(x_ref[...], w_ref[...], preferred_element_type=jnp.float32)
    h_ref[0] = hb
    ap = jnp.dot(hb, att_ref[...], preferred_element_type=jnp.float32)

    @pl.when(half == 0)
    def _():
        apair_ref[...] = ap

    @pl.when(half == 1)
    def _():
        acc = apair_ref[...] + ap
        apair_ref[...] = acc
        amax_ref[0, 0, :] = jnp.max(acc, axis=0)


def _projection(features, W, att2):
    n = features.shape[0]
    nb = n // _ROWS_B
    return pl.pallas_call(
        _proj_body,
        grid=(nb, 2),
        in_specs=[
            pl.BlockSpec((_ROWS_B, D), lambda i, h: (i, 0)),
            pl.BlockSpec((D, DH), lambda i, h: (0, h)),
            pl.BlockSpec((DH, 2), lambda i, h: (h, 0)),
        ],
        out_specs=[
            pl.BlockSpec((1, _ROWS_B, DH), lambda i, h: (h, i, 0)),
            pl.BlockSpec((_ROWS_B, 2), lambda i, h: (i, 0)),
            pl.BlockSpec((1, 1, 2), lambda i, h: (i, 0, 0)),
        ],
        out_shape=[
            jax.ShapeDtypeStruct((2, n, DH), jnp.float32),
            jax.ShapeDtypeStruct((n, 2), jnp.float32),
            jax.ShapeDtypeStruct((nb, 1, 2), jnp.float32),
        ],
    )(features, W, att2)


_VR_A = EW_A // 16
NPH = NP // 2
_RED = NPH // 8
CAP = 2304


def _edge_logits_body(src_hbm, dst_hbm, asrc_hbm, adst_hbm, mvec_hbm,
                      ex_hbm, den_hbm, elist_hbm,
                      as_v, ad_v, mv_v, src_v, dst_v, ex_v, ldenom,
                      el0, el1, el2,
                      shared_den, racc_v, rtmp_v):
    c = lax.axis_index("c")
    s = lax.axis_index("s")
    wid = c * NS + s
    base = wid * EW_A

    pltpu.sync_copy(asrc_hbm, as_v)
    pltpu.sync_copy(adst_hbm, ad_v)
    pltpu.sync_copy(mvec_hbm, mv_v)
    pltpu.sync_copy(src_hbm.at[pl.ds(base, EW_A)], src_v)
    pltpu.sync_copy(dst_hbm.at[pl.ds(base, EW_A)], dst_v)

    zeros16 = jnp.zeros((16,), jnp.float32)

    def zbody(i, _):
        ldenom[pl.ds(i * 16, 16)] = zeros16
        return 0

    lax.fori_loop(0, NP // 16, zbody, 0)

    zeros16i = jnp.zeros((16,), jnp.int32)

    def zlbody(i, _):
        sl = pl.ds(i * 16, 16)
        el0[sl] = zeros16i
        el1[sl] = zeros16i
        el2[sl] = zeros16i
        return 0

    lax.fori_loop(0, CAP // 16, zlbody, 0)

    mv = mv_v[...]
    iota16 = lax.iota(jnp.int32, 16)

    def ebody(t, carry):
        w0, w1, w2 = carry
        sl = pl.ds(t * 16, 16)
        si = src_v[sl]
        di = dst_v[sl]
        di_cl = jnp.minimum(di, NP - 1)
        a_s = plsc.load_gather(as_v, [si])
        a_d = plsc.load_gather(ad_v, [di_cl])
        sm = a_s + a_d
        e = jnp.where(sm > 0, sm, 0.2 * sm)
        ex = jnp.exp(e - mv)
        ex_v[sl] = ex
        plsc.addupdate_scatter(ldenom, [di_cl], ex)
        rel = iota16 + t * 16
        m0 = di < HR
        m1 = (di >= HR) & (di < 2 * HR)
        m2 = (di >= 2 * HR) & (di < NPB)
        for m, el, w_idx in ((m0, el0, 0), (m1, el1, 1), (m2, el2, 2)):
            pc = plsc.cumsum(m.astype(jnp.int32))
            w = (w0, w1, w2)[w_idx]
            plsc.store_scatter(el, [pc + (w - 1)], rel, mask=m)
            nw = w + jnp.max(pc)
            if w_idx == 0:
                w0 = nw
            elif w_idx == 1:
                w1 = nw
            else:
                w2 = nw
        return (w0, w1, w2)

    z32 = jnp.int32(0)
    w0, w1, w2 = lax.fori_loop(0, _VR_A, ebody, (z32, z32, z32))

    del w0, w1, w2
    pltpu.sync_copy(ex_v, ex_hbm.at[pl.ds(base, EW_A)])
    for r_idx, el in enumerate((el0, el1, el2)):
        pltpu.sync_copy(el, elist_hbm.at[wid, pl.ds(r_idx * CAP, CAP)])

    for half in range(2):
        pltpu.sync_copy(ldenom.at[pl.ds(half * NPH, NPH)],
                        shared_den.at[pl.ds(s * NPH, NPH)])
        plsc.subcore_barrier()
        active = (s < 8) if half == 0 else (s >= 8)

        @pl.when(active)
        def _():
            rbase = (s - 8 * half) * _RED
            pltpu.sync_copy(shared_den.at[pl.ds(rbase, _RED)], racc_v)

            def rbody(k, _):
                pltpu.sync_copy(
                    shared_den.at[pl.ds(k * NPH + rbase, _RED)], rtmp_v)

                def abody(q, _):
                    sl = pl.ds(q * 16, 16)
                    racc_v[sl] = racc_v[sl] + rtmp_v[sl]
                    return 0

                lax.fori_loop(0, _RED // 16, abody, 0)
                return 0

            lax.fori_loop(1, NS, rbody, 0)
            pltpu.sync_copy(racc_v,
                            den_hbm.at[c, pl.ds(half * NPH + rbase, _RED)])

        plsc.subcore_barrier()


def _edge_logits(src_pad, dst_pad, asrc_pad, adst_pad, mvec):
    mesh = plsc.VectorSubcoreMesh(
        core_axis_name="c", subcore_axis_name="s", num_cores=NC,
        num_subcores=NS)
    f = pl.kernel(
        _edge_logits_body,
        out_type=[
            jax.ShapeDtypeStruct((E_PAD,), jnp.float32),
            jax.ShapeDtypeStruct((NC, NP), jnp.float32),
            jax.ShapeDtypeStruct((NW, NPASS * CAP), jnp.int32),
        ],
        mesh=mesh,
        compiler_params=pltpu.CompilerParams(needs_layout_passes=False),
        scratch_types=[
            pltpu.VMEM((NP,), jnp.float32),
            pltpu.VMEM((NP,), jnp.float32),
            pltpu.VMEM((16,), jnp.float32),
            pltpu.VMEM((EW_A,), jnp.int32),
            pltpu.VMEM((EW_A,), jnp.int32),
            pltpu.VMEM((EW_A,), jnp.float32),
            pltpu.VMEM((NP,), jnp.float32),
            pltpu.VMEM((CAP,), jnp.int32),
            pltpu.VMEM((CAP,), jnp.int32),
            pltpu.VMEM((CAP,), jnp.int32),
            pltpu.VMEM_SHARED((NS * NPH,), jnp.float32),
            pltpu.VMEM((_RED,), jnp.float32),
            pltpu.VMEM((_RED,), jnp.float32),
        ],
    )
    return f(src_pad, dst_pad, asrc_pad, adst_pad, mvec)


_EW_B = E_PAD // NS
_ACC_R = 3584
_IR_B = _ACC_R // NS
_ER_B = HR // NS


def _msg_body(src_hbm, dst_hbm, ex_hbm, den_hbm, h_hbm, bias_hbm,
              elist_hbm,
              emb_hbm,
              inv_v, src_v, dst_v, ex_v, alpha_v,
              ra, elv, sidx, abuf, dst_g, bias_v, acc_sh, sem_a):
    c = lax.axis_index("c")
    s = lax.axis_index("s")

    pltpu.sync_copy(den_hbm.at[0], inv_v)
    pltpu.sync_copy(den_hbm.at[1], alpha_v)

    def dbody(i, _):
        sl = pl.ds(i * 16, 16)
        inv_v[sl] = 1.0 / (inv_v[sl] + alpha_v[sl] + 1e-16)
        return 0

    lax.fori_loop(0, NP // 16, dbody, 0)

    base = s * _EW_B
    pltpu.sync_copy(src_hbm.at[pl.ds(base, _EW_B)], src_v)
    pltpu.sync_copy(dst_hbm.at[pl.ds(base, _EW_B)], dst_v)
    pltpu.sync_copy(ex_hbm.at[pl.ds(base, _EW_B)], ex_v)

    def abody(t, carry):
        n0, n1, n2 = carry
        sl = pl.ds(t * 16, 16)
        di = dst_v[sl]
        iv = plsc.load_gather(inv_v, [jnp.minimum(di, NP - 1)])
        alpha_v[sl] = ex_v[sl] * iv
        m0 = di < HR
        m1 = (di >= HR) & (di < 2 * HR)
        m2 = (di >= 2 * HR) & (di < NPB)
        n0 = n0 + jnp.max(plsc.all_reduce_population_count(m0))
        n1 = n1 + jnp.max(plsc.all_reduce_population_count(m1))
        n2 = n2 + jnp.max(plsc.all_reduce_population_count(m2))
        return (n0, n1, n2)

    z32 = jnp.int32(0)
    nvr = EW_A // 16
    cnts_a0 = lax.fori_loop(0, nvr, abody, (z32, z32, z32))
    cnts_a1 = lax.fori_loop(nvr, 2 * nvr, abody, (z32, z32, z32))
    counts = (cnts_a0, cnts_a1)

    hc = h_hbm.at[c]
    pltpu.sync_copy(bias_hbm.at[c], bias_v)
    bvs = [bias_v[pl.ds(16 * k, 16)] for k in range(8)]
    trash = HR + s
    iota16 = lax.iota(jnp.int32, 16)

    for r in range(NPASS):
        def ibody(row, _):
            for k in range(8):
                ra[row, pl.ds(16 * k, 16)] = bvs[k]
            return 0

        lax.fori_loop(0, 128, ibody, 0)
        i0 = s * _IR_B
        pltpu.sync_copy(ra, acc_sh.at[pl.ds(i0, 128)])
        pltpu.sync_copy(ra.at[pl.ds(0, _IR_B - 128)],
                        acc_sh.at[pl.ds(i0 + 128, _IR_B - 128)])
        plsc.subcore_barrier()

        for a in range(2):
            pltpu.sync_copy(
                elist_hbm.at[2 * s + a, pl.ds(r * CAP, CAP)], elv.at[a])

        for a in range(2):
            cnt = counts[a][r]
            cnt_b = jnp.full((16,), cnt, jnp.int32)
            abase = a * EW_A

            def gbody(g, _):
                gb = g * 128
                for k in range(8):
                    sl = pl.ds(16 * k, 16)
                    er = elv[a, pl.ds(gb + 16 * k, 16)] + abase
                    sidx[sl] = plsc.load_gather(src_v, [er])
                    abuf[sl] = plsc.load_gather(alpha_v, [er])
                    dl = plsc.load_gather(dst_v, [er]) - (r * HR)
                    valid = (iota16 + (gb + 16 * k)) < cnt_b
                    dst_g[sl] = jnp.where(valid, dl, trash)
                pltpu.async_copy(hc.at[sidx], ra, sem_a).wait()

                def pbody(j, _):
                    av = plsc.load_gather(
                        abuf, [jnp.full((16,), j, jnp.int32)])
                    for k in range(8):
                        sl = pl.ds(16 * k, 16)
                        ra[j, sl] = ra[j, sl] * av
                    return 0

                lax.fori_loop(0, 128, pbody, 0)
                pltpu.sync_copy(ra, acc_sh.at[dst_g], add=True)
                return 0

            lax.fori_loop(0, CAP // 128, gbody, 0)

        plsc.subcore_barrier()

        for t, chunk in ((0, 128), (128, _ER_B - 128)):
            r0 = s * _ER_B + t
            pltpu.sync_copy(acc_sh.at[pl.ds(r0, chunk)],
                            ra.at[pl.ds(0, chunk)])

            def erow(row, _):
                for k in range(8):
                    sl = pl.ds(16 * k, 16)
                    x = ra[row, sl]
                    ra[row, sl] = jnp.where(
                        x > 0, x, jnp.exp(jnp.minimum(x, 0.0)) - 1.0)
                return 0

            lax.fori_loop(0, chunk, erow, 0)
            pltpu.sync_copy(ra.at[pl.ds(0, chunk)],
                            emb_hbm.at[c].at[pl.ds(r * HR + r0, chunk)])
        plsc.subcore_barrier()


def _message_pass(src_pad, dst_pad, ex_pad, denom_p, h_pair, bias2, elist):
    mesh = plsc.VectorSubcoreMesh(
        core_axis_name="c", subcore_axis_name="s", num_cores=NC,
        num_subcores=NS)
    f = pl.kernel(
        _msg_body,
        out_type=jax.ShapeDtypeStruct((NC, NPB, DH), jnp.float32),
        mesh=mesh,
        compiler_params=pltpu.CompilerParams(needs_layout_passes=False),
        scratch_types=[
            pltpu.VMEM((NP,), jnp.float32),
            pltpu.VMEM((_EW_B,), jnp.int32),
            pltpu.VMEM((_EW_B,), jnp.int32),
            pltpu.VMEM((_EW_B,), jnp.float32),
            pltpu.VMEM((_EW_B,), jnp.float32),
            pltpu.VMEM((128, DH), jnp.float32),
            pltpu.VMEM((2, CAP), jnp.int32),
            pltpu.VMEM((128,), jnp.int32),
            pltpu.VMEM((128,), jnp.float32),
            pltpu.VMEM((128,), jnp.int32),
            pltpu.VMEM((DH,), jnp.float32),
            pltpu.VMEM_SHARED((_ACC_R, DH), jnp.float32),
            pltpu.SemaphoreType.DMA,
        ],
    )
    return f(src_pad, dst_pad, ex_pad, denom_p, h_pair, bias2, elist)


SE = 278528
_EW_C = SE // NS
_NG_C = _EW_C // 128


def _score_body(hh_hbm, tt_hbm, emb_hbm,
                pp_hbm,
                hh_v, tt_v, ha, ta, hb, tb, pout, sem_a, sem_b):
    c = lax.axis_index("c")
    s = lax.axis_index("s")
    base = s * _EW_C

    pltpu.sync_copy(hh_hbm.at[pl.ds(base, _EW_C)], hh_v)
    pltpu.sync_copy(tt_hbm.at[pl.ds(base, _EW_C)], tt_v)

    ec = emb_hbm.at[c]

    def issue(g, hbuf, tbuf, sem):
        pltpu.async_copy(ec.at[hh_v.at[pl.ds(g * 128, 128)]], hbuf, sem)
        pltpu.async_copy(ec.at[tt_v.at[pl.ds(g * 128, 128)]], tbuf, sem)

    issue(0, ha, ta, sem_a)
    issue(1, hb, tb, sem_b)

    def process(g, hbuf, tbuf):
        def pbody(j, _):
            acc = hbuf[j, pl.ds(0, 16)] * tbuf[j, pl.ds(0, 16)]
            for k in range(1, 8):
                sl = pl.ds(16 * k, 16)
                acc = acc + hbuf[j, sl] * tbuf[j, sl]
            pout[j, :] = acc
            return 0

        lax.fori_loop(0, 128, pbody, 0)
        pltpu.sync_copy(pout, pp_hbm.at[c].at[pl.ds(base + g * 128, 128)])

    def outer(t, _):
        for b in range(2):
            hbuf, tbuf, sem = (ha, ta, sem_a) if b == 0 else (hb, tb, sem_b)
            g = t * 2 + b
            pltpu.make_async_copy(ec.at[pl.ds(0, 128)], hbuf, sem).wait()
            pltpu.make_async_copy(ec.at[pl.ds(0, 128)], tbuf, sem).wait()
            process(g, hbuf, tbuf)

            @pl.when(g + 2 < _NG_C)
            def _():
                issue(g + 2, hbuf, tbuf, sem)

        return 0

    lax.fori_loop(0, _NG_C // 2, outer, 0)


def _score(heads, tails, emb_pair):
    mesh = plsc.VectorSubcoreMesh(
        core_axis_name="c", subcore_axis_name="s", num_cores=NC,
        num_subcores=NS)
    f = pl.kernel(
        _score_body,
        out_type=jax.ShapeDtypeStruct((NC, SE, 16), jnp.float32),
        mesh=mesh,
        compiler_params=pltpu.CompilerParams(needs_layout_passes=False),
        scratch_types=[
            pltpu.VMEM((_EW_C,), jnp.int32),
            pltpu.VMEM((_EW_C,), jnp.int32),
            pltpu.VMEM((128, DH), jnp.float32),
            pltpu.VMEM((128, DH), jnp.float32),
            pltpu.VMEM((128, DH), jnp.float32),
            pltpu.VMEM((128, DH), jnp.float32),
            pltpu.VMEM((128, 16), jnp.float32),
            pltpu.SemaphoreType.DMA,
            pltpu.SemaphoreType.DMA,
        ],
    )
    return f(heads, tails, emb_pair)


_LROWS = 2048


def _loss_body(pp_ref, g_ref, out_ref):
    pid = pl.program_id(0)
    x = pp_ref[0] + pp_ref[1]
    s = jnp.dot(x, g_ref[...], preferred_element_type=jnp.float32)
    s = jnp.where(pid == 0, s, -s)
    ls = jnp.minimum(s, 0.0) - jnp.log1p(jnp.exp(-jnp.abs(s)))
    bsum = jnp.sum(ls).reshape(1, 1)
    nblk = pl.num_programs(0)
    acc = jnp.where(pid == 0, jnp.zeros((1, 1), jnp.float32),
                    out_ref[...]) + bsum
    out_ref[...] = jnp.where(pid == nblk - 1, -acc / 16384.0, acc)


def _loss(pp3):
    nrows = pp3.shape[1]
    grid = nrows // _LROWS
    gmat = np.zeros((128, 8), np.float32)
    for l in range(128):
        gmat[l, l // 16] = 1.0
    return pl.pallas_call(
        _loss_body,
        grid=(grid,),
        in_specs=[
            pl.BlockSpec((NC, _LROWS, 128), lambda i: (0, i, 0)),
            pl.BlockSpec((128, 8), lambda i: (0, 0)),
        ],
        out_specs=pl.BlockSpec((1, 1), lambda i: (0, 0)),
        out_shape=jax.ShapeDtypeStruct((1, 1), jnp.float32),
    )(pp3, jnp.asarray(gmat))


def kernel(features, edge_index, pos_edge_index, neg_edge_index,
           neg_sample_size, W, att_src, att_dst, bias):
    n = features.shape[0]
    att2 = jnp.stack([att_src, att_dst], axis=1)
    h_pair, apair, amax = _projection(features, W, att2)

    mx = jnp.max(amax, axis=(0, 1))
    msum = mx[0] + mx[1]
    M = jnp.where(msum > 0, msum, 0.2 * msum)

    src = edge_index[0]
    dst = edge_index[1]
    n_edges = src.shape[0]

    src_pad = jnp.concatenate(
        [src, jnp.zeros((E_PAD - n_edges,), jnp.int32)])
    dst_pad = jnp.concatenate(
        [dst, jnp.full((E_PAD - n_edges,), PAD_DST, jnp.int32)])
    zpad = jnp.zeros((NP - n,), jnp.float32)
    asrc_pad = jnp.concatenate([apair[:, 0], zpad])
    adst_pad = jnp.concatenate([apair[:, 1], zpad])
    mvec = jnp.full((16,), M, jnp.float32)

    ex_pad, denom_p, elist = _edge_logits(
        src_pad, dst_pad, asrc_pad, adst_pad, mvec)

    bias2 = bias.reshape(NC, DH)
    emb_pair = _message_pass(src_pad, dst_pad, ex_pad, denom_p, h_pair, bias2,
                             elist)
    emb = jnp.concatenate([emb_pair[0, :n], emb_pair[1, :n]], axis=1)

    sedges = jnp.concatenate([pos_edge_index, neg_edge_index], axis=1)
    pp = _score(sedges[0], sedges[1], emb_pair)
    loss = _loss(pp.reshape(NC, SE * 16 // 128, 128))[0, 0]
    return emb, loss


# speedup vs baseline: 5.6874x; 1.1551x over previous
import functools

import jax
import jax.numpy as jnp
import numpy as np
from jax import lax
from jax.experimental import pallas as pl
from jax.experimental.pallas import tpu as pltpu
from jax.experimental.pallas import tpu_sc as plsc

N_NODES = 10000
D = 256
DH = 128
DQ = 64

NC = 2
NS = 16
NW = NC * NS

NP = 10240
E_PAD = 163840
EW_A = E_PAD // NW
NPASS = 3
HR = 3456
NPB = NPASS * HR
PAD_DST = 16384


_ROWS_B = 1000


def _proj_body(x_ref, w_ref, att_ref, h_ref, apair_ref, amax_ref):
    half = pl.program_id(1)
    hb = jnp.dot(x_ref[...], w_ref[...], preferred_element_type=jnp.float32)
    h_ref[0] = hb
    ap = jnp.dot(hb, att_ref[...], preferred_element_type=jnp.float32)

    @pl.when(half == 0)
    def _():
        apair_ref[...] = ap

    @pl.when(half == 1)
    def _():
        acc = apair_ref[...] + ap
        apair_ref[...] = acc
        amax_ref[0, 0, :] = jnp.max(acc, axis=0)


def _projection(features, W, att2):
    n = features.shape[0]
    nb = n // _ROWS_B
    return pl.pallas_call(
        _proj_body,
        grid=(nb, 2),
        in_specs=[
            pl.BlockSpec((_ROWS_B, D), lambda i, h: (i, 0)),
            pl.BlockSpec((D, DH), lambda i, h: (0, h)),
            pl.BlockSpec((DH, 2), lambda i, h: (h, 0)),
        ],
        out_specs=[
            pl.BlockSpec((1, _ROWS_B, DH), lambda i, h: (h, i, 0)),
            pl.BlockSpec((_ROWS_B, 2), lambda i, h: (i, 0)),
            pl.BlockSpec((1, 1, 2), lambda i, h: (i, 0, 0)),
        ],
        out_shape=[
            jax.ShapeDtypeStruct((2, n, DH), jnp.float32),
            jax.ShapeDtypeStruct((n, 2), jnp.float32),
            jax.ShapeDtypeStruct((nb, 1, 2), jnp.float32),
        ],
    )(features, W, att2)


_VR_A = EW_A // 16
NPH = NP // 2
_RED = NPH // 8
CAP = 2304


def _edge_logits_body(src_hbm, dst_hbm, asrc_hbm, adst_hbm, mvec_hbm,
                      ex_hbm, den_hbm, elist_hbm,
                      as_v, ad_v, mv_v, src_v, dst_v, ex_v, ldenom,
                      el0, el1, el2,
                      shared_den, racc_v, rtmp_v):
    c = lax.axis_index("c")
    s = lax.axis_index("s")
    wid = c * NS + s
    base = wid * EW_A

    pltpu.sync_copy(asrc_hbm, as_v)
    pltpu.sync_copy(adst_hbm, ad_v)
    pltpu.sync_copy(mvec_hbm, mv_v)
    pltpu.sync_copy(src_hbm.at[pl.ds(base, EW_A)], src_v)
    pltpu.sync_copy(dst_hbm.at[pl.ds(base, EW_A)], dst_v)

    zeros16 = jnp.zeros((16,), jnp.float32)

    def zbody(i, _):
        ldenom[pl.ds(i * 16, 16)] = zeros16
        return 0

    lax.fori_loop(0, NP // 16, zbody, 0)

    zeros16i = jnp.zeros((16,), jnp.int32)

    def zlbody(i, _):
        sl = pl.ds(i * 16, 16)
        el0[sl] = zeros16i
        el1[sl] = zeros16i
        el2[sl] = zeros16i
        return 0

    lax.fori_loop(0, CAP // 16, zlbody, 0)

    mv = mv_v[...]
    iota16 = lax.iota(jnp.int32, 16)

    def ebody(t, carry):
        w0, w1, w2 = carry
        sl = pl.ds(t * 16, 16)
        si = src_v[sl]
        di = dst_v[sl]
        di_cl = jnp.minimum(di, NP - 1)
        a_s = plsc.load_gather(as_v, [si])
        a_d = plsc.load_gather(ad_v, [di_cl])
        sm = a_s + a_d
        e = jnp.where(sm > 0, sm, 0.2 * sm)
        ex = jnp.exp(e - mv)
        ex_v[sl] = ex
        plsc.addupdate_scatter(ldenom, [di_cl], ex)
        rel = iota16 + t * 16
        m0 = di < HR
        m1 = (di >= HR) & (di < 2 * HR)
        m2 = (di >= 2 * HR) & (di < NPB)
        for m, el, w_idx in ((m0, el0, 0), (m1, el1, 1), (m2, el2, 2)):
            pc = plsc.cumsum(m.astype(jnp.int32))
            w = (w0, w1, w2)[w_idx]
            plsc.store_scatter(el, [pc + (w - 1)], rel, mask=m)
            nw = w + jnp.max(pc)
            if w_idx == 0:
                w0 = nw
            elif w_idx == 1:
                w1 = nw
            else:
                w2 = nw
        return (w0, w1, w2)

    z32 = jnp.int32(0)
    w0, w1, w2 = lax.fori_loop(0, _VR_A, ebody, (z32, z32, z32))

    del w0, w1, w2
    pltpu.sync_copy(ex_v, ex_hbm.at[pl.ds(base, EW_A)])
    for r_idx, el in enumerate((el0, el1, el2)):
        pltpu.sync_copy(el, elist_hbm.at[wid, pl.ds(r_idx * CAP, CAP)])

    for half in range(2):
        pltpu.sync_copy(ldenom.at[pl.ds(half * NPH, NPH)],
                        shared_den.at[pl.ds(s * NPH, NPH)])
        plsc.subcore_barrier()
        active = (s < 8) if half == 0 else (s >= 8)

        @pl.when(active)
        def _():
            rbase = (s - 8 * half) * _RED
            pltpu.sync_copy(shared_den.at[pl.ds(rbase, _RED)], racc_v)

            def rbody(k, _):
                pltpu.sync_copy(
                    shared_den.at[pl.ds(k * NPH + rbase, _RED)], rtmp_v)

                def abody(q, _):
                    sl = pl.ds(q * 16, 16)
                    racc_v[sl] = racc_v[sl] + rtmp_v[sl]
                    return 0

                lax.fori_loop(0, _RED // 16, abody, 0)
                return 0

            lax.fori_loop(1, NS, rbody, 0)
            pltpu.sync_copy(racc_v,
                            den_hbm.at[c, pl.ds(half * NPH + rbase, _RED)])

        plsc.subcore_barrier()


def _edge_logits(src_pad, dst_pad, asrc_pad, adst_pad, mvec):
    mesh = plsc.VectorSubcoreMesh(
        core_axis_name="c", subcore_axis_name="s", num_cores=NC,
        num_subcores=NS)
    f = pl.kernel(
        _edge_logits_body,
        out_type=[
            jax.ShapeDtypeStruct((E_PAD,), jnp.float32),
            jax.ShapeDtypeStruct((NC, NP), jnp.float32),
            jax.ShapeDtypeStruct((NW, NPASS * CAP), jnp.int32),
        ],
        mesh=mesh,
        compiler_params=pltpu.CompilerParams(needs_layout_passes=False),
        scratch_types=[
            pltpu.VMEM((NP,), jnp.float32),
            pltpu.VMEM((NP,), jnp.float32),
            pltpu.VMEM((16,), jnp.float32),
            pltpu.VMEM((EW_A,), jnp.int32),
            pltpu.VMEM((EW_A,), jnp.int32),
            pltpu.VMEM((EW_A,), jnp.float32),
            pltpu.VMEM((NP,), jnp.float32),
            pltpu.VMEM((CAP,), jnp.int32),
            pltpu.VMEM((CAP,), jnp.int32),
            pltpu.VMEM((CAP,), jnp.int32),
            pltpu.VMEM_SHARED((NS * NPH,), jnp.float32),
            pltpu.VMEM((_RED,), jnp.float32),
            pltpu.VMEM((_RED,), jnp.float32),
        ],
    )
    return f(src_pad, dst_pad, asrc_pad, adst_pad, mvec)


_EW_B = E_PAD // NS
_ACC_R = 3584
_IR_B = _ACC_R // NS
_ER_B = HR // NS


def _msg_body(src_hbm, dst_hbm, ex_hbm, den_hbm, h_hbm, bias_hbm,
              elist_hbm,
              emb_hbm,
              inv_v, src_v, dst_v, ex_v, alpha_v,
              ra, rb, elv, sidx, abuf, dst_g, sidx2, abuf2, dst_g2,
              bias_v, acc_sh, sem_a, sem_b):
    c = lax.axis_index("c")
    s = lax.axis_index("s")

    pltpu.sync_copy(den_hbm.at[0], inv_v)
    pltpu.sync_copy(den_hbm.at[1], alpha_v)

    def dbody(i, _):
        sl = pl.ds(i * 16, 16)
        inv_v[sl] = 1.0 / (inv_v[sl] + alpha_v[sl] + 1e-16)
        return 0

    lax.fori_loop(0, NP // 16, dbody, 0)

    base = s * _EW_B
    pltpu.sync_copy(src_hbm.at[pl.ds(base, _EW_B)], src_v)
    pltpu.sync_copy(dst_hbm.at[pl.ds(base, _EW_B)], dst_v)
    pltpu.sync_copy(ex_hbm.at[pl.ds(base, _EW_B)], ex_v)

    def abody(t, carry):
        n0, n1, n2 = carry
        sl = pl.ds(t * 16, 16)
        di = dst_v[sl]
        iv = plsc.load_gather(inv_v, [jnp.minimum(di, NP - 1)])
        alpha_v[sl] = ex_v[sl] * iv
        m0 = di < HR
        m1 = (di >= HR) & (di < 2 * HR)
        m2 = (di >= 2 * HR) & (di < NPB)
        n0 = n0 + jnp.max(plsc.all_reduce_population_count(m0))
        n1 = n1 + jnp.max(plsc.all_reduce_population_count(m1))
        n2 = n2 + jnp.max(plsc.all_reduce_population_count(m2))
        return (n0, n1, n2)

    z32 = jnp.int32(0)
    nvr = EW_A // 16
    cnts_a0 = lax.fori_loop(0, nvr, abody, (z32, z32, z32))
    cnts_a1 = lax.fori_loop(nvr, 2 * nvr, abody, (z32, z32, z32))
    counts = (cnts_a0, cnts_a1)

    hc = h_hbm.at[c]
    pltpu.sync_copy(bias_hbm.at[c], bias_v)
    bvs = [bias_v[pl.ds(16 * k, 16)] for k in range(8)]
    trash = HR + s
    iota16 = lax.iota(jnp.int32, 16)

    for r in range(NPASS):
        def ibody(row, _):
            for k in range(8):
                ra[row, pl.ds(16 * k, 16)] = bvs[k]
            return 0

        lax.fori_loop(0, 128, ibody, 0)
        i0 = s * _IR_B
        pltpu.sync_copy(ra, acc_sh.at[pl.ds(i0, 128)])
        pltpu.sync_copy(ra.at[pl.ds(0, _IR_B - 128)],
                        acc_sh.at[pl.ds(i0 + 128, _IR_B - 128)])
        plsc.subcore_barrier()

        for a in range(2):
            pltpu.sync_copy(
                elist_hbm.at[2 * s + a, pl.ds(r * CAP, CAP)], elv.at[a])

        for a in range(2):
            cnt = counts[a][r]
            cnt_b = jnp.full((16,), cnt, jnp.int32)
            abase = a * EW_A

            def build(g, sx, ab, dg):
                gb = g * 128
                for k in range(8):
                    sl = pl.ds(16 * k, 16)
                    er = elv[a, pl.ds(gb + 16 * k, 16)] + abase
                    sx[sl] = plsc.load_gather(src_v, [er])
                    ab[sl] = plsc.load_gather(alpha_v, [er])
                    dl = plsc.load_gather(dst_v, [er]) - (r * HR)
                    valid = (iota16 + (gb + 16 * k)) < cnt_b
                    dg[sl] = jnp.where(valid, dl, trash)

            def issue(sx, r_buf, sem):
                pltpu.async_copy(hc.at[sx], r_buf, sem)

            build(0, sidx, abuf, dst_g)
            issue(sidx, ra, sem_a)
            build(1, sidx2, abuf2, dst_g2)
            issue(sidx2, rb, sem_b)

            def gouter(t, _):
                for b in range(2):
                    r_buf, sem = (ra, sem_a) if b == 0 else (rb, sem_b)
                    sx = sidx if b == 0 else sidx2
                    ab = abuf if b == 0 else abuf2
                    dg = dst_g if b == 0 else dst_g2
                    g = t * 2 + b
                    pltpu.make_async_copy(
                        hc.at[pl.ds(0, 128)], r_buf, sem).wait()

                    def pbody(j, _):
                        av = plsc.load_gather(
                            ab, [jnp.full((16,), j, jnp.int32)])
                        for k in range(8):
                            sl = pl.ds(16 * k, 16)
                            r_buf[j, sl] = r_buf[j, sl] * av
                        return 0

                    lax.fori_loop(0, 128, pbody, 0)
                    pltpu.sync_copy(r_buf, acc_sh.at[dg], add=True)

                    @pl.when(g + 2 < CAP // 128)
                    def _():
                        build(g + 2, sx, ab, dg)
                        issue(sx, r_buf, sem)

                return 0

            lax.fori_loop(0, CAP // 256, gouter, 0)

        plsc.subcore_barrier()

        for t, chunk in ((0, 128), (128, _ER_B - 128)):
            r0 = s * _ER_B + t
            pltpu.sync_copy(acc_sh.at[pl.ds(r0, chunk)],
                            ra.at[pl.ds(0, chunk)])

            def erow(row, _):
                for k in range(8):
                    sl = pl.ds(16 * k, 16)
                    x = ra[row, sl]
                    ra[row, sl] = jnp.where(
                        x > 0, x, jnp.exp(jnp.minimum(x, 0.0)) - 1.0)
                return 0

            lax.fori_loop(0, chunk, erow, 0)
            pltpu.sync_copy(ra.at[pl.ds(0, chunk)],
                            emb_hbm.at[c].at[pl.ds(r * HR + r0, chunk)])
        plsc.subcore_barrier()


def _message_pass(src_pad, dst_pad, ex_pad, denom_p, h_pair, bias2, elist):
    mesh = plsc.VectorSubcoreMesh(
        core_axis_name="c", subcore_axis_name="s", num_cores=NC,
        num_subcores=NS)
    f = pl.kernel(
        _msg_body,
        out_type=jax.ShapeDtypeStruct((NC, NPB, DH), jnp.float32),
        mesh=mesh,
        compiler_params=pltpu.CompilerParams(needs_layout_passes=False),
        scratch_types=[
            pltpu.VMEM((NP,), jnp.float32),
            pltpu.VMEM((_EW_B,), jnp.int32),
            pltpu.VMEM((_EW_B,), jnp.int32),
            pltpu.VMEM((_EW_B,), jnp.float32),
            pltpu.VMEM((_EW_B,), jnp.float32),
            pltpu.VMEM((128, DH), jnp.float32),
            pltpu.VMEM((128, DH), jnp.float32),
            pltpu.VMEM((2, CAP), jnp.int32),
            pltpu.VMEM((128,), jnp.int32),
            pltpu.VMEM((128,), jnp.float32),
            pltpu.VMEM((128,), jnp.int32),
            pltpu.VMEM((128,), jnp.int32),
            pltpu.VMEM((128,), jnp.float32),
            pltpu.VMEM((128,), jnp.int32),
            pltpu.VMEM((DH,), jnp.float32),
            pltpu.VMEM_SHARED((_ACC_R, DH), jnp.float32),
            pltpu.SemaphoreType.DMA,
            pltpu.SemaphoreType.DMA,
        ],
    )
    return f(src_pad, dst_pad, ex_pad, denom_p, h_pair, bias2, elist)


SE = 278528
_EW_C = SE // NS
_NG_C = _EW_C // 128


def _score_body(hh_hbm, tt_hbm, emb_hbm,
                pp_hbm,
                hh_v, tt_v, ha, ta, hb, tb, pout, sem_a, sem_b):
    c = lax.axis_index("c")
    s = lax.axis_index("s")
    base = s * _EW_C

    pltpu.sync_copy(hh_hbm.at[pl.ds(base, _EW_C)], hh_v)
    pltpu.sync_copy(tt_hbm.at[pl.ds(base, _EW_C)], tt_v)

    ec = emb_hbm.at[c]

    def issue(g, hbuf, tbuf, sem):
        pltpu.async_copy(ec.at[hh_v.at[pl.ds(g * 128, 128)]], hbuf, sem)
        pltpu.async_copy(ec.at[tt_v.at[pl.ds(g * 128, 128)]], tbuf, sem)

    issue(0, ha, ta, sem_a)
    issue(1, hb, tb, sem_b)

    def process(g, hbuf, tbuf):
        def pbody(j, _):
            acc = hbuf[j, pl.ds(0, 16)] * tbuf[j, pl.ds(0, 16)]
            for k in range(1, 8):
                sl = pl.ds(16 * k, 16)
                acc = acc + hbuf[j, sl] * tbuf[j, sl]
            pout[j, :] = acc
            return 0

        lax.fori_loop(0, 128, pbody, 0)
        pltpu.sync_copy(pout, pp_hbm.at[c].at[pl.ds(base + g * 128, 128)])

    def outer(t, _):
        for b in range(2):
            hbuf, tbuf, sem = (ha, ta, sem_a) if b == 0 else (hb, tb, sem_b)
            g = t * 2 + b
            pltpu.make_async_copy(ec.at[pl.ds(0, 128)], hbuf, sem).wait()
            pltpu.make_async_copy(ec.at[pl.ds(0, 128)], tbuf, sem).wait()
            process(g, hbuf, tbuf)

            @pl.when(g + 2 < _NG_C)
            def _():
                issue(g + 2, hbuf, tbuf, sem)

        return 0

    lax.fori_loop(0, _NG_C // 2, outer, 0)


def _score(heads, tails, emb_pair):
    mesh = plsc.VectorSubcoreMesh(
        core_axis_name="c", subcore_axis_name="s", num_cores=NC,
        num_subcores=NS)
    f = pl.kernel(
        _score_body,
        out_type=jax.ShapeDtypeStruct((NC, SE, 16), jnp.float32),
        mesh=mesh,
        compiler_params=pltpu.CompilerParams(needs_layout_passes=False),
        scratch_types=[
            pltpu.VMEM((_EW_C,), jnp.int32),
            pltpu.VMEM((_EW_C,), jnp.int32),
            pltpu.VMEM((128, DH), jnp.float32),
            pltpu.VMEM((128, DH), jnp.float32),
            pltpu.VMEM((128, DH), jnp.float32),
            pltpu.VMEM((128, DH), jnp.float32),
            pltpu.VMEM((128, 16), jnp.float32),
            pltpu.SemaphoreType.DMA,
            pltpu.SemaphoreType.DMA,
        ],
    )
    return f(heads, tails, emb_pair)


_LROWS = 2048


def _loss_body(pp_ref, g_ref, out_ref):
    pid = pl.program_id(0)
    x = pp_ref[0] + pp_ref[1]
    s = jnp.dot(x, g_ref[...], preferred_element_type=jnp.float32)
    s = jnp.where(pid == 0, s, -s)
    ls = jnp.minimum(s, 0.0) - jnp.log1p(jnp.exp(-jnp.abs(s)))
    bsum = jnp.sum(ls).reshape(1, 1)
    nblk = pl.num_programs(0)
    acc = jnp.where(pid == 0, jnp.zeros((1, 1), jnp.float32),
                    out_ref[...]) + bsum
    out_ref[...] = jnp.where(pid == nblk - 1, -acc / 16384.0, acc)


def _loss(pp3):
    nrows = pp3.shape[1]
    grid = nrows // _LROWS
    gmat = np.zeros((128, 8), np.float32)
    for l in range(128):
        gmat[l, l // 16] = 1.0
    return pl.pallas_call(
        _loss_body,
        grid=(grid,),
        in_specs=[
            pl.BlockSpec((NC, _LROWS, 128), lambda i: (0, i, 0)),
            pl.BlockSpec((128, 8), lambda i: (0, 0)),
        ],
        out_specs=pl.BlockSpec((1, 1), lambda i: (0, 0)),
        out_shape=jax.ShapeDtypeStruct((1, 1), jnp.float32),
    )(pp3, jnp.asarray(gmat))


def kernel(features, edge_index, pos_edge_index, neg_edge_index,
           neg_sample_size, W, att_src, att_dst, bias):
    n = features.shape[0]
    att2 = jnp.stack([att_src, att_dst], axis=1)
    h_pair, apair, amax = _projection(features, W, att2)

    mx = jnp.max(amax, axis=(0, 1))
    msum = mx[0] + mx[1]
    M = jnp.where(msum > 0, msum, 0.2 * msum)

    src = edge_index[0]
    dst = edge_index[1]
    n_edges = src.shape[0]

    src_pad = jnp.concatenate(
        [src, jnp.zeros((E_PAD - n_edges,), jnp.int32)])
    dst_pad = jnp.concatenate(
        [dst, jnp.full((E_PAD - n_edges,), PAD_DST, jnp.int32)])
    zpad = jnp.zeros((NP - n,), jnp.float32)
    asrc_pad = jnp.concatenate([apair[:, 0], zpad])
    adst_pad = jnp.concatenate([apair[:, 1], zpad])
    mvec = jnp.full((16,), M, jnp.float32)

    ex_pad, denom_p, elist = _edge_logits(
        src_pad, dst_pad, asrc_pad, adst_pad, mvec)

    bias2 = bias.reshape(NC, DH)
    emb_pair = _message_pass(src_pad, dst_pad, ex_pad, denom_p, h_pair, bias2,
                             elist)
    emb = jnp.concatenate([emb_pair[0, :n], emb_pair[1, :n]], axis=1)

    sedges = jnp.concatenate([pos_edge_index, neg_edge_index], axis=1)
    pp = _score(sedges[0], sedges[1], emb_pair)
    loss = _loss(pp.reshape(NC, SE * 16 // 128, 128))[0, 0]
    return emb, loss
